# baseline (speedup 1.0000x reference)
import functools

import jax
import jax.numpy as jnp
from jax import lax
from jax.experimental import pallas as pl
from jax.experimental.pallas import tpu as pltpu
from jax.experimental.pallas import tpu_sc as plsc

B = 16384
D = 64
V = 1000000
NC = 2
NS = 16
NW = NC * NS
BPW = B // NW
CHUNK = 128
NCH = BPW // CHUNK
L = 16
TBLK = 8192


def _transpose_body(inT_ref, out_ref):
    for dh in range(D // 8):
        strip = inT_ref[pl.ds(dh * 8, 8), :]
        out_ref[:, pl.ds(dh * 8, 8)] = strip.T


_transpose_tc = pl.pallas_call(
    _transpose_body,
    out_shape=jax.ShapeDtypeStruct((V, D), jnp.float32),
    grid=(pl.cdiv(V, TBLK),),
    in_specs=[pl.BlockSpec((D, TBLK), lambda i: (0, i))],
    out_specs=pl.BlockSpec((TBLK, D), lambda i: (i, 0)),
)

_mesh = plsc.VectorSubcoreMesh(core_axis_name="c", subcore_axis_name="s")


@functools.partial(
    pl.kernel,
    out_type=jax.ShapeDtypeStruct((B,), jnp.float32),
    mesh=_mesh,
    compiler_params=pltpu.CompilerParams(
        needs_layout_passes=False, use_tc_tiling_on_sc=False),
    scratch_types=[
        pltpu.VMEM((BPW,), jnp.int32),
        pltpu.VMEM((BPW,), jnp.int32),
        pltpu.VMEM((CHUNK, D), jnp.float32),
        pltpu.VMEM((CHUNK, D), jnp.float32),
        pltpu.VMEM((BPW,), jnp.float32),
        pltpu.VMEM((BPW,), jnp.float32),
        pltpu.VMEM((BPW,), jnp.float32),
        pltpu.SemaphoreType.DMA,
    ],
)
def _mf_kernel(user_hbm, item_hbm, uf_hbm, if_hbm, ub_hbm, ib_hbm, out_hbm,
               uidx_v, iidx_v, urows_v, irows_v, ub_v, ib_v, out_v, sem):
    wid = lax.axis_index("s") * NC + lax.axis_index("c")
    base = wid * BPW

    pltpu.sync_copy(user_hbm.at[pl.ds(base, BPW)], uidx_v)
    pltpu.sync_copy(item_hbm.at[pl.ds(base, BPW)], iidx_v)

    for c in range(NCH):
        s = pl.ds(c * CHUNK, CHUNK)
        pltpu.async_copy(ub_hbm.at[uidx_v.at[s]], ub_v.at[s], sem)
        pltpu.async_copy(ib_hbm.at[iidx_v.at[s]], ib_v.at[s], sem)
    pltpu.make_async_copy(ub_hbm.at[pl.ds(0, BPW)], ub_v, sem).wait()
    pltpu.make_async_copy(ib_hbm.at[pl.ds(0, BPW)], ib_v, sem).wait()

    for c in range(NCH):
        idx_u = uidx_v.at[pl.ds(c * CHUNK, CHUNK)]
        idx_i = iidx_v.at[pl.ds(c * CHUNK, CHUNK)]
        cps = [
            pltpu.async_copy(uf_hbm.at[idx_u], urows_v, sem),
            pltpu.async_copy(if_hbm.at[idx_i], irows_v, sem),
        ]
        for cp in cps:
            cp.wait()

        for g in range(CHUNK // L):
            rows = lax.iota(jnp.int32, L) + g * L
            acc0 = ub_v[pl.ds(c * CHUNK + g * L, L)] + ib_v[
                pl.ds(c * CHUNK + g * L, L)]

            def body(d, acc, rows=rows):
                dd = jnp.full((L,), 0, jnp.int32) + d
                uv = plsc.load_gather(urows_v, [rows, dd])
                iv = plsc.load_gather(irows_v, [rows, dd])
                return acc + uv * iv

            out_v[pl.ds(c * CHUNK + g * L, L)] = lax.fori_loop(
                0, D, body, acc0)

    pltpu.sync_copy(out_v, out_hbm.at[pl.ds(base, BPW)])


def kernel(user, item, user_factors, item_factors, user_biases, item_biases):
    uf_rm = _transpose_tc(user_factors.T)
    if_rm = _transpose_tc(item_factors.T)
    return _mf_kernel(user.astype(jnp.int32), item.astype(jnp.int32),
                      uf_rm, if_rm,
                      user_biases.reshape(-1), item_biases.reshape(-1))


# speedup vs baseline: 8.0630x; 8.0630x over previous
import functools

import jax
import jax.numpy as jnp
from jax import lax
from jax.experimental import pallas as pl
from jax.experimental.pallas import tpu as pltpu
from jax.experimental.pallas import tpu_sc as plsc

B = 16384
D = 64
V = 1000000
NC = 2
NS = 16
NW = NC * NS
BPW = B // NW
L = 16
TBLK = 8192
NBLK = pl.cdiv(V, TBLK)
NPAN = NBLK * (TBLK // 128)
PANW = 8 * 128
DHW = NPAN * PANW
FLAT = 8 * DHW

CHUNKE = 128
NCH = BPW // CHUNKE
CW = CHUNKE * D


def _pack_body(inT_ref, out_ref):
    x = inT_ref[...].reshape(8, 8, TBLK // 128, 128)
    out_ref[...] = x.transpose(0, 2, 1, 3)


_pack_tc = pl.pallas_call(
    _pack_body,
    out_shape=jax.ShapeDtypeStruct((8, NPAN, 8, 128), jnp.float32),
    grid=(NBLK,),
    in_specs=[pl.BlockSpec((D, TBLK), lambda i: (0, i))],
    out_specs=pl.BlockSpec((8, TBLK // 128, 8, 128), lambda i: (0, i, 0, 0)),
)

_mesh = plsc.VectorSubcoreMesh(core_axis_name="c", subcore_axis_name="s")


@functools.partial(
    pl.kernel,
    out_type=jax.ShapeDtypeStruct((B,), jnp.float32),
    mesh=_mesh,
    compiler_params=pltpu.CompilerParams(
        needs_layout_passes=False, use_tc_tiling_on_sc=False),
    scratch_types=[
        pltpu.VMEM((BPW,), jnp.int32),
        pltpu.VMEM((BPW,), jnp.int32),
        pltpu.VMEM((BPW,), jnp.int32),
        pltpu.VMEM((BPW,), jnp.int32),
        pltpu.VMEM((CW,), jnp.int32),
        pltpu.VMEM((CW,), jnp.int32),
        pltpu.VMEM((CW,), jnp.float32),
        pltpu.VMEM((CW,), jnp.float32),
        pltpu.VMEM((BPW,), jnp.float32),
        pltpu.VMEM((BPW,), jnp.float32),
        pltpu.VMEM((BPW,), jnp.float32),
        pltpu.SemaphoreType.DMA,
    ],
)
def _mf_kernel(user_hbm, item_hbm, uflat_hbm, iflat_hbm, ub_hbm, ib_hbm,
               out_hbm, uidx_v, iidx_v, ubase_v, ibase_v, uwidx, iwidx,
               udst, idst, ub_v, ib_v, out_v, sem):
    wid = lax.axis_index("s") * NC + lax.axis_index("c")
    base = wid * BPW

    pltpu.sync_copy(user_hbm.at[pl.ds(base, BPW)], uidx_v)
    pltpu.sync_copy(item_hbm.at[pl.ds(base, BPW)], iidx_v)

    for c in range(BPW // 128):
        s = pl.ds(c * 128, 128)
        pltpu.async_copy(ub_hbm.at[uidx_v.at[s]], ub_v.at[s], sem)
        pltpu.async_copy(ib_hbm.at[iidx_v.at[s]], ib_v.at[s], sem)

    for g in range(BPW // L):
        s = pl.ds(g * L, L)
        u = uidx_v[s]
        ubase_v[s] = (u >> 7) * 1024 + (u & 127)
        v = iidx_v[s]
        ibase_v[s] = (v >> 7) * 1024 + (v & 127)

    pltpu.make_async_copy(ub_hbm.at[pl.ds(0, BPW)], ub_v, sem).wait()
    pltpu.make_async_copy(ib_hbm.at[pl.ds(0, BPW)], ib_v, sem).wait()

    for c in range(NCH):
        for g in range(CHUNKE // L):
            s = pl.ds(c * CHUNKE + g * L, L)
            ub = ubase_v[s]
            ib = ibase_v[s]

            def ibody(d, _, ub=ub, ib=ib, g=g):
                off = (d >> 3) * DHW + (d & 7) * 128
                slot = pl.ds(g * (L * D) + d * L, L)
                uwidx[slot] = ub + off
                iwidx[slot] = ib + off
                return 0

            lax.fori_loop(0, D, ibody, 0)

        cps = [
            pltpu.async_copy(uflat_hbm.at[uwidx], udst, sem),
            pltpu.async_copy(iflat_hbm.at[iwidx], idst, sem),
        ]
        for cp in cps:
            cp.wait()

        for g in range(CHUNKE // L):
            acc0 = ub_v[pl.ds(c * CHUNKE + g * L, L)] + ib_v[
                pl.ds(c * CHUNKE + g * L, L)]

            def body(d, acc, g=g):
                slot = pl.ds(g * (L * D) + d * L, L)
                return acc + udst[slot] * idst[slot]

            out_v[pl.ds(c * CHUNKE + g * L, L)] = lax.fori_loop(
                0, D, body, acc0)

    pltpu.sync_copy(out_v, out_hbm.at[pl.ds(base, BPW)])


def kernel(user, item, user_factors, item_factors, user_biases, item_biases):
    uflat = _pack_tc(user_factors.T).reshape(FLAT)
    iflat = _pack_tc(item_factors.T).reshape(FLAT)
    return _mf_kernel(user.astype(jnp.int32), item.astype(jnp.int32),
                      uflat, iflat,
                      user_biases.reshape(-1), item_biases.reshape(-1))


# speedup vs baseline: 11.0984x; 1.3765x over previous
import functools

import jax
import jax.numpy as jnp
from jax import lax
from jax.experimental import pallas as pl
from jax.experimental.pallas import tpu as pltpu
from jax.experimental.pallas import tpu_sc as plsc

B = 16384
D = 64
DP = D // 2
V = 1000000
NC = 2
NS = 16
NW = NC * NS
BPW = B // NW
L = 16
TBLK = 16384
NBLK = pl.cdiv(V, TBLK)
NPAN = NBLK * (TBLK // 128)
DHW = NPAN * 4 * 128
FLAT = 8 * DHW

CHUNKE = 128
NCH = BPW // CHUNKE
CW = CHUNKE * DP


def _pack_body(inT_ref, out_ref):
    x = inT_ref[...].reshape(8, 4, 2, TBLK // 128, 128)
    bits = jax.lax.bitcast_convert_type(x, jnp.uint32)
    rounded = (bits + jnp.uint32(0x8000)) & jnp.uint32(0xFFFF0000)
    a = rounded[:, :, 0]
    b_ = rounded[:, :, 1] >> jnp.uint32(16)
    w = (a | b_).astype(jnp.int32)
    w = w.transpose(0, 2, 1, 3)
    out_ref[...] = w.reshape(8, (TBLK // 128) * 4, 128)


_pack_tc = pl.pallas_call(
    _pack_body,
    out_shape=jax.ShapeDtypeStruct((8, NPAN * 4, 128), jnp.int32),
    grid=(NBLK,),
    in_specs=[pl.BlockSpec((D, TBLK), lambda i: (0, i))],
    out_specs=pl.BlockSpec((8, (TBLK // 128) * 4, 128), lambda i: (0, i, 0)),
)

_mesh = plsc.VectorSubcoreMesh(core_axis_name="c", subcore_axis_name="s")


@functools.partial(
    pl.kernel,
    out_type=jax.ShapeDtypeStruct((B,), jnp.float32),
    mesh=_mesh,
    compiler_params=pltpu.CompilerParams(
        needs_layout_passes=False, use_tc_tiling_on_sc=False),
    scratch_types=[
        pltpu.VMEM((BPW,), jnp.int32),
        pltpu.VMEM((BPW,), jnp.int32),
        pltpu.VMEM((BPW,), jnp.int32),
        pltpu.VMEM((BPW,), jnp.int32),
        pltpu.VMEM((CW,), jnp.int32),
        pltpu.VMEM((CW,), jnp.int32),
        pltpu.VMEM((CW,), jnp.int32),
        pltpu.VMEM((CW,), jnp.int32),
        pltpu.VMEM((BPW,), jnp.float32),
        pltpu.VMEM((BPW,), jnp.float32),
        pltpu.VMEM((BPW,), jnp.float32),
        pltpu.SemaphoreType.DMA,
    ],
)
def _mf_kernel(user_hbm, item_hbm, uflat_hbm, iflat_hbm, ub_hbm, ib_hbm,
               out_hbm, uidx_v, iidx_v, ubase_v, ibase_v, uwidx, iwidx,
               udst, idst, ub_v, ib_v, out_v, sem):
    wid = lax.axis_index("s") * NC + lax.axis_index("c")
    base = wid * BPW

    pltpu.sync_copy(user_hbm.at[pl.ds(base, BPW)], uidx_v)
    pltpu.sync_copy(item_hbm.at[pl.ds(base, BPW)], iidx_v)

    for c in range(BPW // 128):
        s = pl.ds(c * 128, 128)
        pltpu.async_copy(ub_hbm.at[uidx_v.at[s]], ub_v.at[s], sem)
        pltpu.async_copy(ib_hbm.at[iidx_v.at[s]], ib_v.at[s], sem)

    for g in range(BPW // L):
        s = pl.ds(g * L, L)
        u = uidx_v[s]
        ubase_v[s] = (u >> 7) * 512 + (u & 127)
        v = iidx_v[s]
        ibase_v[s] = (v >> 7) * 512 + (v & 127)

    pltpu.make_async_copy(ub_hbm.at[pl.ds(0, BPW)], ub_v, sem).wait()
    pltpu.make_async_copy(ib_hbm.at[pl.ds(0, BPW)], ib_v, sem).wait()

    for c in range(NCH):
        for g in range(CHUNKE // L):
            s = pl.ds(c * CHUNKE + g * L, L)
            ub = ubase_v[s]
            ib = ibase_v[s]

            def ibody(p, _, ub=ub, ib=ib, g=g):
                off = (p >> 2) * DHW + (p & 3) * 128
                slot = pl.ds(g * (L * DP) + p * L, L)
                uwidx[slot] = ub + off
                iwidx[slot] = ib + off
                return 0

            lax.fori_loop(0, DP, ibody, 0)

        cps = [
            pltpu.async_copy(uflat_hbm.at[uwidx], udst, sem),
            pltpu.async_copy(iflat_hbm.at[iwidx], idst, sem),
        ]
        for cp in cps:
            cp.wait()

        for g in range(CHUNKE // L):
            acc0 = ub_v[pl.ds(c * CHUNKE + g * L, L)] + ib_v[
                pl.ds(c * CHUNKE + g * L, L)]

            def body(p, acc, g=g):
                slot = pl.ds(g * (L * DP) + p * L, L)
                uw = udst[slot]
                iw = idst[slot]
                ua = plsc.bitcast(uw & jnp.int32(-65536), jnp.float32)
                ia = plsc.bitcast(iw & jnp.int32(-65536), jnp.float32)
                ub_f = plsc.bitcast(uw << 16, jnp.float32)
                ib_f = plsc.bitcast(iw << 16, jnp.float32)
                return acc + ua * ia + ub_f * ib_f

            out_v[pl.ds(c * CHUNKE + g * L, L)] = lax.fori_loop(
                0, DP, body, acc0)

    pltpu.sync_copy(out_v, out_hbm.at[pl.ds(base, BPW)])


def kernel(user, item, user_factors, item_factors, user_biases, item_biases):
    uflat = _pack_tc(user_factors.T).reshape(FLAT)
    iflat = _pack_tc(item_factors.T).reshape(FLAT)
    return _mf_kernel(user.astype(jnp.int32), item.astype(jnp.int32),
                      uflat, iflat,
                      user_biases.reshape(-1), item_biases.reshape(-1))


# speedup vs baseline: 11.1337x; 1.0032x over previous
import functools

import jax
import jax.numpy as jnp
from jax import lax
from jax.experimental import pallas as pl
from jax.experimental.pallas import tpu as pltpu
from jax.experimental.pallas import tpu_sc as plsc

B = 16384
D = 64
DP = D // 2
V = 1000000
NC = 2
NS = 16
NW = NC * NS
BPW = B // NW
L = 16
TBLK = 16384
NBLK = pl.cdiv(V, TBLK)
NPAN = NBLK * (TBLK // 128)
DHW = NPAN * 4 * 128
FLAT = 8 * DHW

CHUNKE = 256
NCH = BPW // CHUNKE
CW = CHUNKE * DP


def _pack_body(inT_ref, out_ref):
    x = inT_ref[...].reshape(8, 4, 2, TBLK // 128, 128)
    bits = jax.lax.bitcast_convert_type(x, jnp.uint32)
    rounded = (bits + jnp.uint32(0x8000)) & jnp.uint32(0xFFFF0000)
    a = rounded[:, :, 0]
    b_ = rounded[:, :, 1] >> jnp.uint32(16)
    w = (a | b_).astype(jnp.int32)
    w = w.transpose(0, 2, 1, 3)
    out_ref[...] = w.reshape(8, (TBLK // 128) * 4, 128)


_pack_tc = pl.pallas_call(
    _pack_body,
    out_shape=jax.ShapeDtypeStruct((8, NPAN * 4, 128), jnp.int32),
    grid=(NBLK,),
    in_specs=[pl.BlockSpec((D, TBLK), lambda i: (0, i))],
    out_specs=pl.BlockSpec((8, (TBLK // 128) * 4, 128), lambda i: (0, i, 0)),
)

_mesh = plsc.VectorSubcoreMesh(core_axis_name="c", subcore_axis_name="s")


@functools.partial(
    pl.kernel,
    out_type=jax.ShapeDtypeStruct((B,), jnp.float32),
    mesh=_mesh,
    compiler_params=pltpu.CompilerParams(
        needs_layout_passes=False, use_tc_tiling_on_sc=False),
    scratch_types=[
        pltpu.VMEM((BPW,), jnp.int32),
        pltpu.VMEM((BPW,), jnp.int32),
        pltpu.VMEM((BPW,), jnp.int32),
        pltpu.VMEM((BPW,), jnp.int32),
        pltpu.VMEM((CW,), jnp.int32),
        pltpu.VMEM((CW,), jnp.int32),
        pltpu.VMEM((CW,), jnp.int32),
        pltpu.VMEM((CW,), jnp.int32),
        pltpu.VMEM((BPW,), jnp.float32),
        pltpu.VMEM((BPW,), jnp.float32),
        pltpu.VMEM((BPW,), jnp.float32),
        pltpu.SemaphoreType.DMA,
    ],
)
def _mf_kernel(user_hbm, item_hbm, uflat_hbm, iflat_hbm, ub_hbm, ib_hbm,
               out_hbm, uidx_v, iidx_v, ubase_v, ibase_v, uwidx, iwidx,
               udst, idst, ub_v, ib_v, out_v, sem):
    wid = lax.axis_index("s") * NC + lax.axis_index("c")
    base = wid * BPW

    pltpu.sync_copy(user_hbm.at[pl.ds(base, BPW)], uidx_v)
    pltpu.sync_copy(item_hbm.at[pl.ds(base, BPW)], iidx_v)

    for c in range(BPW // 128):
        s = pl.ds(c * 128, 128)
        pltpu.async_copy(ub_hbm.at[uidx_v.at[s]], ub_v.at[s], sem)
        pltpu.async_copy(ib_hbm.at[iidx_v.at[s]], ib_v.at[s], sem)

    for g in range(BPW // L):
        s = pl.ds(g * L, L)
        u = uidx_v[s]
        ubase_v[s] = (u >> 7) * 512 + (u & 127)
        v = iidx_v[s]
        ibase_v[s] = (v >> 7) * 512 + (v & 127)

    pltpu.make_async_copy(ub_hbm.at[pl.ds(0, BPW)], ub_v, sem).wait()
    pltpu.make_async_copy(ib_hbm.at[pl.ds(0, BPW)], ib_v, sem).wait()

    for c in range(NCH):
        for g in range(CHUNKE // L):
            s = pl.ds(c * CHUNKE + g * L, L)
            ub = ubase_v[s]
            ib = ibase_v[s]

            def ibody(p, _, ub=ub, ib=ib, g=g):
                off = (p >> 2) * DHW + (p & 3) * 128
                slot = pl.ds(g * (L * DP) + p * L, L)
                uwidx[slot] = ub + off
                iwidx[slot] = ib + off
                return 0

            lax.fori_loop(0, DP, ibody, 0)

        cps = [
            pltpu.async_copy(uflat_hbm.at[uwidx], udst, sem),
            pltpu.async_copy(iflat_hbm.at[iwidx], idst, sem),
        ]
        for cp in cps:
            cp.wait()

        for g in range(CHUNKE // L):
            acc0 = ub_v[pl.ds(c * CHUNKE + g * L, L)] + ib_v[
                pl.ds(c * CHUNKE + g * L, L)]

            def body(p, acc, g=g):
                slot = pl.ds(g * (L * DP) + p * L, L)
                uw = udst[slot]
                iw = idst[slot]
                ua = plsc.bitcast(uw & jnp.int32(-65536), jnp.float32)
                ia = plsc.bitcast(iw & jnp.int32(-65536), jnp.float32)
                ub_f = plsc.bitcast(uw << 16, jnp.float32)
                ib_f = plsc.bitcast(iw << 16, jnp.float32)
                return acc + ua * ia + ub_f * ib_f

            out_v[pl.ds(c * CHUNKE + g * L, L)] = lax.fori_loop(
                0, DP, body, acc0)

    pltpu.sync_copy(out_v, out_hbm.at[pl.ds(base, BPW)])


def kernel(user, item, user_factors, item_factors, user_biases, item_biases):
    uflat = _pack_tc(user_factors.T).reshape(FLAT)
    iflat = _pack_tc(item_factors.T).reshape(FLAT)
    return _mf_kernel(user.astype(jnp.int32), item.astype(jnp.int32),
                      uflat, iflat,
                      user_biases.reshape(-1), item_biases.reshape(-1))
